# Initial kernel scaffold; baseline (speedup 1.0000x reference)
import jax
import jax.numpy as jnp
from jax.experimental import pallas as pl


def kernel(x, edge_index, edge_attr, W_l1, b_l1, W_r1, W_l2, b_l2, W_r2, W_c1, b_c1, W_c2, b_c2):
    raise NotImplementedError("write your pallas kernel here")


# speedup vs baseline: 3.1814x; 3.1814x over previous
import functools

import jax
import jax.numpy as jnp
from jax import lax
from jax.experimental import pallas as pl
from jax.experimental.pallas import tpu as pltpu
from jax.experimental.pallas import tpu_sc as plsc

N = 10000
E = 320000
DF = 128
DE = 16
H = 256

NC = 2
NS = 16
CH = 80
NP_ = 10240

_mesh = lambda: plsc.VectorSubcoreMesh(core_axis_name="c", subcore_axis_name="s")


def _make_sc_agg1():
    per_tile = E // (NC * NS)
    nchunk = per_tile // CH
    G1 = 5
    ngroups = nchunk // G1
    rows_pt = NP_ // NS

    out_type = [jax.ShapeDtypeStruct((NP_, 128), jnp.float32),
                jax.ShapeDtypeStruct((NP_, 128), jnp.float32)]
    scratch = [pltpu.VMEM((G1, CH), jnp.int32),
               pltpu.VMEM((G1, CH), jnp.int32),
               pltpu.VMEM((CH, 128), jnp.float32),
               pltpu.VMEM_SHARED((NP_, 128), jnp.float32),
               pltpu.SemaphoreType.DMA]

    def body(x_hbm, src_hbm, dst_hbm, z_hbm,
             p0, p1, srcv, dstv, rows, acc, sem):
        c = lax.axis_index("c")
        s = lax.axis_index("s")
        wid = s * NC + c
        r0 = s * rows_pt
        pltpu.sync_copy(z_hbm.at[pl.ds(r0, rows_pt)], acc.at[pl.ds(r0, rows_pt)])
        plsc.subcore_barrier()

        def group(gg, carry):
            pltpu.sync_copy(src_hbm.at[wid * ngroups + gg], srcv)
            pltpu.sync_copy(dst_hbm.at[wid * ngroups + gg], dstv)

            def step(j, cc):
                pltpu.async_copy(x_hbm.at[srcv.at[j]], rows, sem).wait()
                pltpu.sync_copy(rows, acc.at[dstv.at[j]], add=True)
                return cc

            lax.fori_loop(0, G1, step, 0)
            return carry

        lax.fori_loop(0, ngroups, group, 0)
        plsc.subcore_barrier()
        sl = pl.ds(r0, rows_pt)

        @pl.when(c == 0)
        def _():
            pltpu.sync_copy(acc.at[sl], p0.at[sl])

        @pl.when(c == 1)
        def _():
            pltpu.sync_copy(acc.at[sl], p1.at[sl])

    return pl.kernel(body, out_type=tuple(out_type), mesh=_mesh(),
                     scratch_types=tuple(scratch))


HALF = NP_ // 2


def _make_sc_count():
    per_tile = E // NS
    nchunk = per_tile // CH
    rows_pt = HALF // NS

    G2 = 10
    ngroups = nchunk // G2
    out_type = jax.ShapeDtypeStruct((NC, HALF, 128), jnp.float32)
    scratch = [pltpu.VMEM((G2, CH), jnp.int32),
               pltpu.VMEM((1, CH), jnp.int32),
               pltpu.VMEM((CH, 128), jnp.float32),
               pltpu.VMEM_SHARED((HALF + 64, 128), jnp.float32),
               pltpu.SemaphoreType.DMA]

    def body(dst_hbm, ones_hbm, z_hbm, occ, dstv, cidxv, onesv, cnt, sem):
        c = lax.axis_index("c")
        s = lax.axis_index("s")
        pltpu.sync_copy(ones_hbm, onesv)
        r0 = s * rows_pt
        pltpu.sync_copy(z_hbm.at[pl.ds(r0, rows_pt)], cnt.at[pl.ds(r0, rows_pt)])
        plsc.subcore_barrier()
        base = c * HALF

        def group(gg, carry):
            pltpu.sync_copy(dst_hbm.at[s * ngroups + gg], dstv)

            def step(j, cc):
                for k in range(CH // 16):
                    d16 = dstv[j, pl.ds(k * 16, 16)]
                    dd = d16 - base
                    ok = (dd >= 0) & (dd < HALF)
                    cidxv[0, pl.ds(k * 16, 16)] = jnp.where(
                        ok, dd, jnp.full((16,), HALF, jnp.int32))
                pltpu.sync_copy(onesv, cnt.at[cidxv.at[0]], add=True)
                return cc

            lax.fori_loop(0, G2, step, 0)
            return carry

        lax.fori_loop(0, ngroups, group, 0)
        plsc.subcore_barrier()
        pltpu.sync_copy(cnt.at[pl.ds(r0, rows_pt)],
                        occ.at[c].at[pl.ds(r0, rows_pt)])

    return pl.kernel(body, out_type=out_type, mesh=_mesh(),
                     scratch_types=tuple(scratch))


def _make_sc_agg2():
    per_tile = E // NS
    nchunk = per_tile // CH
    G2 = 10
    ngroups = nchunk // G2
    rows_pt = NP_ // NS

    out_type = [jax.ShapeDtypeStruct((NP_, 128), jnp.float32),
                jax.ShapeDtypeStruct((NP_, 128), jnp.float32)]
    scratch = [pltpu.VMEM((G2, CH), jnp.int32),
               pltpu.VMEM((G2, CH), jnp.int32),
               pltpu.VMEM((CH, 128), jnp.float32),
               pltpu.VMEM_SHARED((NP_, 128), jnp.float32),
               pltpu.SemaphoreType.DMA]

    def body(h_lo, h_hi, src_hbm, dst_hbm, z_hbm,
             out_lo, out_hi, srcv, dstv, rows, acc, sem):
        c = lax.axis_index("c")
        s = lax.axis_index("s")
        r0 = s * rows_pt
        pltpu.sync_copy(z_hbm.at[pl.ds(r0, rows_pt)], acc.at[pl.ds(r0, rows_pt)])
        plsc.subcore_barrier()

        def group(gg, carry):
            pltpu.sync_copy(src_hbm.at[s * ngroups + gg], srcv)
            pltpu.sync_copy(dst_hbm.at[s * ngroups + gg], dstv)

            def step(j, cc):
                @pl.when(c == 0)
                def _():
                    pltpu.async_copy(h_lo.at[srcv.at[j]], rows, sem).wait()

                @pl.when(c == 1)
                def _():
                    pltpu.async_copy(h_hi.at[srcv.at[j]], rows, sem).wait()

                pltpu.sync_copy(rows, acc.at[dstv.at[j]], add=True)
                return cc

            lax.fori_loop(0, G2, step, 0)
            return carry

        lax.fori_loop(0, ngroups, group, 0)
        plsc.subcore_barrier()
        sl = pl.ds(r0, rows_pt)

        @pl.when(c == 0)
        def _():
            pltpu.sync_copy(acc.at[sl], out_lo.at[sl])

        @pl.when(c == 1)
        def _():
            pltpu.sync_copy(acc.at[sl], out_hi.at[sl])

    return pl.kernel(body, out_type=tuple(out_type), mesh=_mesh(),
                     scratch_types=tuple(scratch))


def _make_sc_classify():
    per_tile = E // (NC * NS)
    nchunk = per_tile // CH
    G1 = 5
    ngroups = nchunk // G1

    scratch = [pltpu.VMEM((G1, CH), jnp.int32),
               pltpu.VMEM((G1, CH), jnp.int32),
               pltpu.VMEM((CH, H), jnp.float32),
               pltpu.VMEM((CH, H), jnp.float32),
               pltpu.VMEM((CH, H), jnp.float32),
               pltpu.VMEM((H,), jnp.float32),
               pltpu.VMEM((CH * 16,), jnp.float32),
               pltpu.SemaphoreType.DMA]

    def body(a_hbm, b_hbm, c_hbm, w_hbm, src_hbm, dst_hbm, out_hbm,
             srcv, dstv, ar, br, cr, wv, ov, sem):
        c = lax.axis_index("c")
        s = lax.axis_index("s")
        wid = s * NC + c
        pltpu.sync_copy(w_hbm, wv)

        def group(gg, carry):
            pltpu.sync_copy(src_hbm.at[wid * ngroups + gg], srcv)
            pltpu.sync_copy(dst_hbm.at[wid * ngroups + gg], dstv)

            def step(j, cc2):
                pltpu.async_copy(a_hbm.at[srcv.at[j]], ar, sem).wait()
                pltpu.async_copy(b_hbm.at[dstv.at[j]], br, sem).wait()
                e0 = wid * per_tile + (gg * G1 + j) * CH
                pltpu.sync_copy(c_hbm.at[pl.ds(e0, CH)], cr)
                def edge(e, cc):
                    acc = jnp.zeros((16,), jnp.float32)
                    for g in range(H // 16):
                        sl = pl.ds(g * 16, 16)
                        v = ar[e, sl] + br[e, sl] + cr[e, sl]
                        acc = acc + jnp.maximum(v, 0.0) * wv[sl]
                    ov[pl.ds(e * 16, 16)] = acc
                    return cc

                lax.fori_loop(0, CH, edge, 0)
                pltpu.sync_copy(ov, out_hbm.at[pl.ds(e0 * 16, CH * 16)])
                return cc2

            lax.fori_loop(0, G1, step, 0)
            return carry

        lax.fori_loop(0, ngroups, group, 0)

    return pl.kernel(body, out_type=jax.ShapeDtypeStruct((E * 16,), jnp.float32),
                     mesh=_mesh(), scratch_types=tuple(scratch))


def _tc_lanesum(x1, S, b_c2):
    TB = 5000
    grid = (E // 8) // TB

    def body(xr, sr, b, o):
        o[...] = (jnp.dot(xr[...], sr[...], preferred_element_type=jnp.float32)
                  + b[0, 0])

    return pl.pallas_call(
        body,
        grid=(grid,),
        in_specs=[pl.BlockSpec((TB, 128), lambda i: (i, 0)),
                  pl.BlockSpec((128, 8), lambda i: (0, 0)),
                  pl.BlockSpec((1, 1), lambda i: (0, 0))],
        out_specs=pl.BlockSpec((TB, 8), lambda i: (i, 0)),
        out_shape=jax.ShapeDtypeStruct((E // 8, 8), jnp.float32),
    )(x1, S, b_c2)


def _tc_layer1(p0, p1, chist, x, W_l, b_l, W_r):
    TM = 1024
    grid = NP_ // TM

    def body(p0r, p1r, cr, xr, wl, b, wr, olo, ohi):
        cnt = cr[0, :, 0:1]
        inv = 1.0 / jnp.maximum(cnt, 1.0)
        m = (p0r[...] + p1r[...]) * inv
        h = (jnp.dot(m, wl[...], preferred_element_type=jnp.float32)
             + jnp.dot(xr[...], wr[...], preferred_element_type=jnp.float32)
             + b[...])
        h = jnp.maximum(h, 0.0)
        olo[...] = h[:, 0:128]
        ohi[...] = h[:, 128:256]

    return pl.pallas_call(
        body,
        grid=(grid,),
        in_specs=[pl.BlockSpec((TM, DF), lambda i: (i, 0)),
                  pl.BlockSpec((TM, DF), lambda i: (i, 0)),
                  pl.BlockSpec((1, TM, 128), lambda i: (i // (HALF // 1024), i % (HALF // 1024), 0)),
                  pl.BlockSpec((TM, DF), lambda i: (i, 0)),
                  pl.BlockSpec((DF, H), lambda i: (0, 0)),
                  pl.BlockSpec((1, H), lambda i: (0, 0)),
                  pl.BlockSpec((DF, H), lambda i: (0, 0))],
        out_specs=[pl.BlockSpec((TM, 128), lambda i: (i, 0)),
                   pl.BlockSpec((TM, 128), lambda i: (i, 0))],
        out_shape=[jax.ShapeDtypeStruct((NP_, 128), jnp.float32),
                   jax.ShapeDtypeStruct((NP_, 128), jnp.float32)],
    )(p0, p1, chist, x, W_l, b_l, W_r)


def _tc_layer2(agg_lo, agg_hi, chist, h_lo, h_hi, W_l, b_l, W_r, Wca, Wcb):
    TM = 1024
    grid = NP_ // TM

    def body(alo, ahi, cr, hlo, hhi, wl, b, wr, wca, wcb, oa, ob):
        cnt = cr[0, :, 0:1]
        inv = 1.0 / jnp.maximum(cnt, 1.0)
        h = (jnp.dot(alo[...] * inv, wl[0:128], preferred_element_type=jnp.float32)
             + jnp.dot(ahi[...] * inv, wl[128:256], preferred_element_type=jnp.float32)
             + jnp.dot(hlo[...], wr[0:128], preferred_element_type=jnp.float32)
             + jnp.dot(hhi[...], wr[128:256], preferred_element_type=jnp.float32)
             + b[...])
        h = jnp.maximum(h, 0.0)
        oa[...] = jnp.dot(h, wca[...], preferred_element_type=jnp.float32)
        ob[...] = jnp.dot(h, wcb[...], preferred_element_type=jnp.float32)

    return pl.pallas_call(
        body,
        grid=(grid,),
        in_specs=[pl.BlockSpec((TM, 128), lambda i: (i, 0)),
                  pl.BlockSpec((TM, 128), lambda i: (i, 0)),
                  pl.BlockSpec((1, TM, 128), lambda i: (i // (HALF // 1024), i % (HALF // 1024), 0)),
                  pl.BlockSpec((TM, 128), lambda i: (i, 0)),
                  pl.BlockSpec((TM, 128), lambda i: (i, 0)),
                  pl.BlockSpec((H, H), lambda i: (0, 0)),
                  pl.BlockSpec((1, H), lambda i: (0, 0)),
                  pl.BlockSpec((H, H), lambda i: (0, 0)),
                  pl.BlockSpec((H, H), lambda i: (0, 0)),
                  pl.BlockSpec((H, H), lambda i: (0, 0))],
        out_specs=[pl.BlockSpec((TM, H), lambda i: (i, 0)),
                   pl.BlockSpec((TM, H), lambda i: (i, 0))],
        out_shape=[jax.ShapeDtypeStruct((NP_, H), jnp.float32),
                   jax.ShapeDtypeStruct((NP_, H), jnp.float32)],
    )(agg_lo, agg_hi, chist, h_lo, h_hi, W_l, b_l, W_r, Wca, Wcb)


def _tc_edge_c(edge_attr, Wce, b_c1):
    TE = 4000
    grid = E // TE

    def body(ea, w, b, o):
        o[...] = jnp.dot(ea[...], w[...], preferred_element_type=jnp.float32) + b[...]

    return pl.pallas_call(
        body,
        grid=(grid,),
        in_specs=[pl.BlockSpec((TE, DE), lambda i: (i, 0)),
                  pl.BlockSpec((DE, H), lambda i: (0, 0)),
                  pl.BlockSpec((1, H), lambda i: (0, 0))],
        out_specs=pl.BlockSpec((TE, H), lambda i: (i, 0)),
        out_shape=jax.ShapeDtypeStruct((E, H), jnp.float32),
    )(edge_attr, Wce, b_c1)


def kernel(x, edge_index, edge_attr, W_l1, b_l1, W_r1, W_l2, b_l2, W_r2,
           W_c1, b_c1, W_c2, b_c2):
    src16 = edge_index[0].reshape(NS * 25, 10, CH)
    dst16 = edge_index[1].reshape(NS * 25, 10, CH)
    src32 = edge_index[0].reshape(NC * NS * 25, 5, CH)
    dst32 = edge_index[1].reshape(NC * NS * 25, 5, CH)
    z128 = jnp.zeros((NP_, 128), jnp.float32)
    ones128 = jnp.ones((CH, 128), jnp.float32)

    agg1 = _make_sc_agg1()
    p0, p1 = agg1(x, src32, dst32, z128)
    occ = _make_sc_count()(dst16, ones128, z128)

    h1_lo, h1_hi = _tc_layer1(p0, p1, occ, x,
                              W_l1, b_l1.reshape(1, H), W_r1)

    agg2 = _make_sc_agg2()
    s2_lo, s2_hi = agg2(h1_lo, h1_hi, src16, dst16, z128)

    A, B = _tc_layer2(s2_lo, s2_hi, occ, h1_lo, h1_hi,
                      W_l2, b_l2.reshape(1, H), W_r2,
                      W_c1[:H], W_c1[H:2 * H])

    C = _tc_edge_c(edge_attr, W_c1[2 * H:], b_c1.reshape(1, H))

    classify = _make_sc_classify()
    out1 = classify(A, B, C, W_c2.reshape(H), src32, dst32)

    S = jnp.kron(jnp.eye(8, dtype=jnp.float32), jnp.ones((16, 1), jnp.float32))
    sums = _tc_lanesum(out1.reshape(E // 8, 128), S, b_c2.reshape(1, 1))
    return sums.reshape(E)


# speedup vs baseline: 4.2341x; 1.3309x over previous
import jax
import jax.numpy as jnp
from jax import lax
from jax.experimental import pallas as pl
from jax.experimental.pallas import tpu as pltpu
from jax.experimental.pallas import tpu_sc as plsc

N = 10000
E = 320000
DF = 128
DE = 16
H = 256

NC = 2
NS = 16
CH = 80
NP_ = 10112

_mesh = lambda: plsc.VectorSubcoreMesh(core_axis_name="c", subcore_axis_name="s")


def _make_sc_agg1():
    per_tile = E // (NC * NS)
    nchunk = per_tile // CH
    G = 5
    ngroups = nchunk // G
    rows_pt = NP_ // NS

    out_type = [jax.ShapeDtypeStruct((NP_, 128), jnp.float32),
                jax.ShapeDtypeStruct((NP_, 128), jnp.float32)]
    scratch = [pltpu.VMEM((G, CH), jnp.int32),
               pltpu.VMEM((G, CH), jnp.int32),
               pltpu.VMEM((CH, 128), jnp.float32),
               pltpu.VMEM((CH, 128), jnp.float32),
               pltpu.VMEM_SHARED((NP_, 128), jnp.float32),
               pltpu.SemaphoreType.DMA,
               pltpu.SemaphoreType.DMA]

    def body(x_hbm, src_hbm, dst_hbm, z_hbm,
             p0, p1, srcv, dstv, rows0, rows1, acc, semg, sems):
        c = lax.axis_index("c")
        s = lax.axis_index("s")
        wid = s * NC + c
        r0 = s * rows_pt
        pltpu.sync_copy(z_hbm.at[pl.ds(r0, rows_pt)], acc.at[pl.ds(r0, rows_pt)])
        plsc.subcore_barrier()
        rows = [rows0, rows1]

        def group(gg, carry):
            pltpu.sync_copy(src_hbm.at[wid * ngroups + gg], srcv)
            pltpu.sync_copy(dst_hbm.at[wid * ngroups + gg], dstv)
            gh = pltpu.async_copy(x_hbm.at[srcv.at[0]], rows[0], semg)
            sh = [None] * G
            for j in range(G):
                gh.wait()
                sh[j] = pltpu.async_copy(rows[j % 2], acc.at[dstv.at[j]],
                                         sems, add=True)
                if j + 1 < G:
                    if j >= 1:
                        sh[j - 1].wait()
                    gh = pltpu.async_copy(x_hbm.at[srcv.at[j + 1]],
                                          rows[(j + 1) % 2], semg)
            sh[G - 2].wait()
            sh[G - 1].wait()
            return carry

        lax.fori_loop(0, ngroups, group, 0)
        plsc.subcore_barrier()
        sl = pl.ds(r0, rows_pt)

        @pl.when(c == 0)
        def _():
            pltpu.sync_copy(acc.at[sl], p0.at[sl])

        @pl.when(c == 1)
        def _():
            pltpu.sync_copy(acc.at[sl], p1.at[sl])

    return pl.kernel(body, out_type=tuple(out_type), mesh=_mesh(),
                     scratch_types=tuple(scratch))


def _make_sc_agg2():
    per_tile = E // NS
    nchunk = per_tile // CH
    G = 5
    ngroups = nchunk // G
    rows_pt = NP_ // NS

    out_type = [jax.ShapeDtypeStruct((NP_, 128), jnp.float32),
                jax.ShapeDtypeStruct((NP_, 128), jnp.float32)]
    scratch = [pltpu.VMEM((G, CH), jnp.int32),
               pltpu.VMEM((G, CH), jnp.int32),
               pltpu.VMEM((CH, 128), jnp.float32),
               pltpu.VMEM((CH, 128), jnp.float32),
               pltpu.VMEM_SHARED((NP_, 128), jnp.float32),
               pltpu.SemaphoreType.DMA,
               pltpu.SemaphoreType.DMA]

    def body(h_lo, h_hi, src_hbm, dst_hbm, z_hbm,
             out_lo, out_hi, srcv, dstv, rows0, rows1, acc, semg, sems):
        c = lax.axis_index("c")
        s = lax.axis_index("s")
        r0 = s * rows_pt
        pltpu.sync_copy(z_hbm.at[pl.ds(r0, rows_pt)], acc.at[pl.ds(r0, rows_pt)])
        plsc.subcore_barrier()
        rows = [rows0, rows1]

        def gather(j, buf):
            @pl.when(c == 0)
            def _():
                pltpu.async_copy(h_lo.at[srcv.at[j]], buf, semg)

            @pl.when(c == 1)
            def _():
                pltpu.async_copy(h_hi.at[srcv.at[j]], buf, semg)

            return pltpu.make_async_copy(h_lo.at[srcv.at[j]], buf, semg)

        def group(gg, carry):
            pltpu.sync_copy(src_hbm.at[s * ngroups + gg], srcv)
            pltpu.sync_copy(dst_hbm.at[s * ngroups + gg], dstv)
            gh = gather(0, rows[0])
            sh = [None] * G
            for j in range(G):
                gh.wait()
                sh[j] = pltpu.async_copy(rows[j % 2], acc.at[dstv.at[j]],
                                         sems, add=True)
                if j + 1 < G:
                    if j >= 1:
                        sh[j - 1].wait()
                    gh = gather(j + 1, rows[(j + 1) % 2])
            sh[G - 2].wait()
            sh[G - 1].wait()
            return carry

        lax.fori_loop(0, ngroups, group, 0)
        plsc.subcore_barrier()
        sl = pl.ds(r0, rows_pt)

        @pl.when(c == 0)
        def _():
            pltpu.sync_copy(acc.at[sl], out_lo.at[sl])

        @pl.when(c == 1)
        def _():
            pltpu.sync_copy(acc.at[sl], out_hi.at[sl])

    return pl.kernel(body, out_type=tuple(out_type), mesh=_mesh(),
                     scratch_types=tuple(scratch))


def _make_sc_count():
    per_tile = E // (NC * NS)
    nchunk = per_tile // CH
    G = 5
    ngroups = nchunk // G
    rows_pt = NP_ // NS

    out_type = jax.ShapeDtypeStruct((NC, NP_, 128), jnp.float32)
    scratch = [pltpu.VMEM((G, CH), jnp.int32),
               pltpu.VMEM((CH, 128), jnp.float32),
               pltpu.VMEM_SHARED((NP_, 128), jnp.float32),
               pltpu.SemaphoreType.DMA]

    def body(dst_hbm, ones_hbm, z_hbm, occ, dstv, onesv, cnt, sems):
        c = lax.axis_index("c")
        s = lax.axis_index("s")
        wid = s * NC + c
        pltpu.sync_copy(ones_hbm, onesv)
        r0 = s * rows_pt
        pltpu.sync_copy(z_hbm.at[pl.ds(r0, rows_pt)], cnt.at[pl.ds(r0, rows_pt)])
        plsc.subcore_barrier()

        def group(gg, carry):
            pltpu.sync_copy(dst_hbm.at[wid * ngroups + gg], dstv)
            sh = [None] * G
            for j in range(G):
                sh[j] = pltpu.async_copy(onesv, cnt.at[dstv.at[j]],
                                         sems, add=True)
            for j in range(G):
                sh[j].wait()
            return carry

        lax.fori_loop(0, ngroups, group, 0)
        plsc.subcore_barrier()
        pltpu.sync_copy(cnt.at[pl.ds(r0, rows_pt)],
                        occ.at[c].at[pl.ds(r0, rows_pt)])

    return pl.kernel(body, out_type=out_type, mesh=_mesh(),
                     scratch_types=tuple(scratch))


def _make_sc_classify():
    per_tile = E // (NC * NS)
    nchunk = per_tile // CH
    G = 5
    ngroups = nchunk // G

    scratch = [pltpu.VMEM((G, CH), jnp.int32),
               pltpu.VMEM((G, CH), jnp.int32),
               pltpu.VMEM((CH, H), jnp.float32),
               pltpu.VMEM((CH, H), jnp.float32),
               pltpu.VMEM((CH, H), jnp.float32),
               pltpu.VMEM((CH, H), jnp.float32),
               pltpu.VMEM((CH, H), jnp.float32),
               pltpu.VMEM((H,), jnp.float32),
               pltpu.VMEM((CH * 16,), jnp.float32),
               pltpu.VMEM((CH * 16,), jnp.float32),
               pltpu.SemaphoreType.DMA,
               pltpu.SemaphoreType.DMA]

    def body(a_hbm, b_hbm, c_hbm, w_hbm, src_hbm, dst_hbm, out_hbm,
             srcv, dstv, ar0, br0, ar1, br1, crv, wv, ov0, ov1,
             semi, semo):
        c = lax.axis_index("c")
        s = lax.axis_index("s")
        wid = s * NC + c
        pltpu.sync_copy(w_hbm, wv)
        ar = [ar0, ar1]
        br = [br0, br1]
        ov = [ov0, ov1]

        def fire(gg, j):
            b = j % 2
            ha = pltpu.async_copy(a_hbm.at[srcv.at[j]], ar[b], semi)
            hb = pltpu.async_copy(b_hbm.at[dstv.at[j]], br[b], semi)
            return (ha, hb)

        def group(gg, carry):
            pltpu.sync_copy(src_hbm.at[wid * ngroups + gg], srcv)
            pltpu.sync_copy(dst_hbm.at[wid * ngroups + gg], dstv)
            h = fire(gg, 0)
            oh = [None] * G
            for j in range(G):
                b = j % 2
                e0 = wid * per_tile + (gg * G + j) * CH
                pltpu.sync_copy(c_hbm.at[pl.ds(e0, CH)], crv)
                for x in h:
                    x.wait()
                if j + 1 < G:
                    h = fire(gg, j + 1)
                if j >= 2:
                    oh[j - 2].wait()

                def edge(e, cc):
                    acc = jnp.zeros((16,), jnp.float32)
                    for g in range(H // 16):
                        sl = pl.ds(g * 16, 16)
                        v = ar[b][e, sl] + br[b][e, sl] + crv[e, sl]
                        acc = acc + jnp.maximum(v, 0.0) * wv[sl]
                    ov[b][pl.ds(e * 16, 16)] = acc
                    return cc

                lax.fori_loop(0, CH, edge, 0)
                oh[j] = pltpu.async_copy(ov[b],
                                         out_hbm.at[pl.ds(e0 * 16, CH * 16)],
                                         semo)
            oh[G - 2].wait()
            oh[G - 1].wait()
            return carry

        lax.fori_loop(0, ngroups, group, 0)

    return pl.kernel(body, out_type=jax.ShapeDtypeStruct((E * 16,), jnp.float32),
                     mesh=_mesh(), scratch_types=tuple(scratch))


def _tc_layer1(p0, p1, occ, x, W_l, b_l, W_r):
    TM = 632
    grid = NP_ // TM

    def body(p0r, p1r, cr, xr, wl, b, wr, olo, ohi):
        cnt = cr[0, :, 0:1] + cr[1, :, 0:1]
        inv = 1.0 / jnp.maximum(cnt, 1.0)
        m = (p0r[...] + p1r[...]) * inv
        h = (jnp.dot(m, wl[...], preferred_element_type=jnp.float32)
             + jnp.dot(xr[...], wr[...], preferred_element_type=jnp.float32)
             + b[...])
        h = jnp.maximum(h, 0.0)
        olo[...] = h[:, 0:128]
        ohi[...] = h[:, 128:256]

    return pl.pallas_call(
        body,
        grid=(grid,),
        in_specs=[pl.BlockSpec((TM, DF), lambda i: (i, 0)),
                  pl.BlockSpec((TM, DF), lambda i: (i, 0)),
                  pl.BlockSpec((NC, TM, 128), lambda i: (0, i, 0)),
                  pl.BlockSpec((TM, DF), lambda i: (i, 0)),
                  pl.BlockSpec((DF, H), lambda i: (0, 0)),
                  pl.BlockSpec((1, H), lambda i: (0, 0)),
                  pl.BlockSpec((DF, H), lambda i: (0, 0))],
        out_specs=[pl.BlockSpec((TM, 128), lambda i: (i, 0)),
                   pl.BlockSpec((TM, 128), lambda i: (i, 0))],
        out_shape=[jax.ShapeDtypeStruct((NP_, 128), jnp.float32),
                   jax.ShapeDtypeStruct((NP_, 128), jnp.float32)],
    )(p0, p1, occ, x, W_l, b_l, W_r)


def _tc_layer2(agg_lo, agg_hi, occ, h_lo, h_hi, W_l, b_l, W_r, Wca, Wcb):
    TM = 632
    grid = NP_ // TM

    def body(alo, ahi, cr, hlo, hhi, wl, b, wr, wca, wcb, oa, ob):
        cnt = cr[0, :, 0:1] + cr[1, :, 0:1]
        inv = 1.0 / jnp.maximum(cnt, 1.0)
        h = (jnp.dot(alo[...] * inv, wl[0:128], preferred_element_type=jnp.float32)
             + jnp.dot(ahi[...] * inv, wl[128:256], preferred_element_type=jnp.float32)
             + jnp.dot(hlo[...], wr[0:128], preferred_element_type=jnp.float32)
             + jnp.dot(hhi[...], wr[128:256], preferred_element_type=jnp.float32)
             + b[...])
        h = jnp.maximum(h, 0.0)
        oa[...] = jnp.dot(h, wca[...], preferred_element_type=jnp.float32)
        ob[...] = jnp.dot(h, wcb[...], preferred_element_type=jnp.float32)

    return pl.pallas_call(
        body,
        grid=(grid,),
        in_specs=[pl.BlockSpec((TM, 128), lambda i: (i, 0)),
                  pl.BlockSpec((TM, 128), lambda i: (i, 0)),
                  pl.BlockSpec((NC, TM, 128), lambda i: (0, i, 0)),
                  pl.BlockSpec((TM, 128), lambda i: (i, 0)),
                  pl.BlockSpec((TM, 128), lambda i: (i, 0)),
                  pl.BlockSpec((H, H), lambda i: (0, 0)),
                  pl.BlockSpec((1, H), lambda i: (0, 0)),
                  pl.BlockSpec((H, H), lambda i: (0, 0)),
                  pl.BlockSpec((H, H), lambda i: (0, 0)),
                  pl.BlockSpec((H, H), lambda i: (0, 0))],
        out_specs=[pl.BlockSpec((TM, H), lambda i: (i, 0)),
                   pl.BlockSpec((TM, H), lambda i: (i, 0))],
        out_shape=[jax.ShapeDtypeStruct((NP_, H), jnp.float32),
                   jax.ShapeDtypeStruct((NP_, H), jnp.float32)],
    )(agg_lo, agg_hi, occ, h_lo, h_hi, W_l, b_l, W_r, Wca, Wcb)


def _tc_edge_c(edge_attr, Wce, b_c1):
    TE = 4000
    grid = E // TE

    def body(ea, w, b, o):
        o[...] = jnp.dot(ea[...], w[...], preferred_element_type=jnp.float32) + b[...]

    return pl.pallas_call(
        body,
        grid=(grid,),
        in_specs=[pl.BlockSpec((TE, DE), lambda i: (i, 0)),
                  pl.BlockSpec((DE, H), lambda i: (0, 0)),
                  pl.BlockSpec((1, H), lambda i: (0, 0))],
        out_specs=pl.BlockSpec((TE, H), lambda i: (i, 0)),
        out_shape=jax.ShapeDtypeStruct((E, H), jnp.float32),
    )(edge_attr, Wce, b_c1)


def _tc_lanesum(x1, S, b_c2):
    TB = 5000
    grid = (E // 8) // TB

    def body(xr, sr, b, o):
        o[...] = (jnp.dot(xr[...], sr[...], preferred_element_type=jnp.float32)
                  + b[0, 0])

    return pl.pallas_call(
        body,
        grid=(grid,),
        in_specs=[pl.BlockSpec((TB, 128), lambda i: (i, 0)),
                  pl.BlockSpec((128, 8), lambda i: (0, 0)),
                  pl.BlockSpec((1, 1), lambda i: (0, 0))],
        out_specs=pl.BlockSpec((TB, 8), lambda i: (i, 0)),
        out_shape=jax.ShapeDtypeStruct((E // 8, 8), jnp.float32),
    )(x1, S, b_c2)


def kernel(x, edge_index, edge_attr, W_l1, b_l1, W_r1, W_l2, b_l2, W_r2,
           W_c1, b_c1, W_c2, b_c2):
    src16 = edge_index[0].reshape(NS * 50, 5, CH)
    dst16 = edge_index[1].reshape(NS * 50, 5, CH)
    src32 = edge_index[0].reshape(NC * NS * 25, 5, CH)
    dst32 = edge_index[1].reshape(NC * NS * 25, 5, CH)
    z128 = jnp.zeros((NP_, 128), jnp.float32)
    ones128 = jnp.ones((CH, 128), jnp.float32)

    agg1 = _make_sc_agg1()
    p0, p1 = agg1(x, src32, dst32, z128)
    occ = _make_sc_count()(dst32, ones128, z128)

    h1_lo, h1_hi = _tc_layer1(p0, p1, occ, x,
                              W_l1, b_l1.reshape(1, H), W_r1)

    agg2 = _make_sc_agg2()
    s2_lo, s2_hi = agg2(h1_lo, h1_hi, src16, dst16, z128)

    A, B = _tc_layer2(s2_lo, s2_hi, occ, h1_lo, h1_hi,
                      W_l2, b_l2.reshape(1, H), W_r2,
                      W_c1[:H], W_c1[H:2 * H])

    C = _tc_edge_c(edge_attr, W_c1[2 * H:], b_c1.reshape(1, H))

    classify = _make_sc_classify()
    out1 = classify(A, B, C, W_c2.reshape(H), src32, dst32)

    S = jnp.kron(jnp.eye(8, dtype=jnp.float32), jnp.ones((16, 1), jnp.float32))
    sums = _tc_lanesum(out1.reshape(E // 8, 128), S, b_c2.reshape(1, 1))
    return sums.reshape(E)
